# baseline (speedup 1.0000x reference)
import functools

import jax
import jax.numpy as jnp
from jax import lax
from jax.experimental import pallas as pl
from jax.experimental.pallas import tpu as pltpu
from jax.experimental.pallas import tpu_sc as plsc

B, S, D, DFF, E, K = 2, 2048, 1024, 4096, 8, 2
T = B * S
A = T * K
TILE = 512
PAD_N = A + E * TILE
NT = PAD_N // TILE
CH = 512
NCH = DFF // CH

NC, NS = 2, 16
NW = NC * NS

GBLK = 1024


def _gate_body(x_ref, wg_ref, idx_ref, w_ref):
    logits = lax.dot_general(x_ref[...], wg_ref[...],
                             (((1,), (1,)), ((), ())),
                             preferred_element_type=jnp.float32)
    neg = jnp.full((GBLK, 1), -jnp.inf, jnp.float32)
    m1, i1 = neg, jnp.zeros((GBLK, 1), jnp.int32)
    for e in range(E):
        v = logits[:, e:e + 1]
        upd = v > m1
        m1 = jnp.where(upd, v, m1)
        i1 = jnp.where(upd, e, i1)
    m2, i2 = neg, jnp.zeros((GBLK, 1), jnp.int32)
    for e in range(E):
        v = logits[:, e:e + 1]
        upd = jnp.logical_and(v > m2, i1 != e)
        m2 = jnp.where(upd, v, m2)
        i2 = jnp.where(upd, e, i2)
    e2 = jnp.exp(m2 - m1)
    w1 = 1.0 / (1.0 + e2)
    idx_ref[...] = jnp.concatenate([i1, i2], axis=1)
    w_ref[...] = jnp.concatenate([w1, 1.0 - w1], axis=1)


def _gate(x2d, Wg):
    return pl.pallas_call(
        _gate_body,
        grid=(T // GBLK,),
        in_specs=[
            pl.BlockSpec((GBLK, D), lambda i: (i, 0)),
            pl.BlockSpec((E, D), lambda i: (0, 0)),
        ],
        out_specs=[
            pl.BlockSpec((GBLK, K), lambda i: (i, 0)),
            pl.BlockSpec((GBLK, K), lambda i: (i, 0)),
        ],
        out_shape=[
            jax.ShapeDtypeStruct((T, K), jnp.int32),
            jax.ShapeDtypeStruct((T, K), jnp.float32),
        ],
    )(x2d, Wg)


def _dispatch(topi, topw):
    ids = topi.reshape(-1)
    wts = topw.reshape(-1)
    onehot = (ids[:, None] == jnp.arange(E, dtype=jnp.int32)[None, :])
    csum = jnp.cumsum(onehot.astype(jnp.int32), axis=0)
    counts = csum[A - 1]
    pc = ((counts + TILE - 1) // TILE) * TILE
    pcc = jnp.cumsum(pc)
    poff = pcc - pc
    rank = jnp.take_along_axis(csum, ids[:, None], axis=1)[:, 0] - 1
    dest = jnp.take(poff, ids) + rank
    ar = jnp.arange(A, dtype=jnp.int32)
    gidx = (jnp.arange(PAD_N, dtype=jnp.int32) % T).at[dest].set(ar // K)
    gw = jnp.zeros((PAD_N,), jnp.float32).at[dest].set(wts)
    p0, p1 = dest[0::2], dest[1::2]
    tile_start = jnp.arange(NT, dtype=jnp.int32) * TILE
    tile_e = jnp.minimum(
        jnp.searchsorted(pcc, tile_start, side="right").astype(jnp.int32),
        E - 1)
    tile_act = (tile_start < pcc[E - 1]).astype(jnp.int32)
    chg = (tile_e[1:] != tile_e[:-1]).astype(jnp.int32)
    tile_ord = jnp.cumsum(jnp.concatenate([jnp.zeros((1,), jnp.int32), chg]))
    nxt_pos = jnp.searchsorted(tile_e, tile_e, side="right").astype(jnp.int32)
    tile_hn = (nxt_pos < NT).astype(jnp.int32)
    tile_nxt = jnp.take(tile_e, jnp.minimum(nxt_pos, NT - 1))
    return (gidx, gw, p0, p1, tile_e, tile_act, tile_ord, tile_nxt, tile_hn)


_G_CHUNK = 32


def _gather_body(n_rows, tab_hbm, idx_hbm, out_hbm,
                 i0, i1, r0, r1, sem):
    per_w = n_rows // NW
    nch = per_w // _G_CHUNK
    wid = lax.axis_index("s") * NC + lax.axis_index("c")
    base = wid * per_w
    bufs = ((i0, r0), (i1, r1))

    pltpu.sync_copy(idx_hbm.at[pl.ds(base, _G_CHUNK)], i0)
    cp = pltpu.async_copy(tab_hbm.at[i0], r0, sem)
    for c in range(nch):
        _, rb = bufs[c % 2]
        if c + 1 < nch:
            inx, rnx = bufs[(c + 1) % 2]
            pltpu.sync_copy(
                idx_hbm.at[pl.ds(base + (c + 1) * _G_CHUNK, _G_CHUNK)], inx)
            cpn = pltpu.async_copy(tab_hbm.at[inx], rnx, sem)
        cp.wait()
        pltpu.sync_copy(rb, out_hbm.at[pl.ds(base + c * _G_CHUNK, _G_CHUNK)])
        if c + 1 < nch:
            cp = cpn


@functools.cache
def _sc_gather_kernel(n_rows):
    return pl.kernel(
        functools.partial(_gather_body, n_rows),
        name=f"sc_row_gather_{n_rows}",
        out_type=jax.ShapeDtypeStruct((n_rows, D), jnp.float32),
        mesh=plsc.VectorSubcoreMesh(core_axis_name="c", subcore_axis_name="s",
                                    num_cores=NC, num_subcores=NS),
        scratch_types=[
            pltpu.VMEM((_G_CHUNK,), jnp.int32),
            pltpu.VMEM((_G_CHUNK,), jnp.int32),
            pltpu.VMEM((_G_CHUNK, D), jnp.float32),
            pltpu.VMEM((_G_CHUNK, D), jnp.float32),
            pltpu.SemaphoreType.DMA,
        ],
    )


def _sc_gather(tab, idx, n_rows):
    return _sc_gather_kernel(n_rows)(tab, idx)


def _ffn_body(te_ref, ta_ref, ord_ref, nxt_ref, hn_ref,
              xg_ref, wcat_hbm, b1_ref, b2_ref, b3_ref,
              gw_ref, out_ref, wbuf, sems):
    i = pl.program_id(0)
    act = ta_ref[i] == 1
    slot = ord_ref[i] % 2
    prev_ord = ord_ref[jnp.maximum(i - 1, 0)]
    is_first = jnp.logical_or(i == 0, ord_ref[i] != prev_ord)

    def start(e_idx, s):
        pltpu.make_async_copy(wcat_hbm.at[e_idx], wbuf.at[s],
                              sems.at[s]).start()

    def wait(s):
        pltpu.make_async_copy(wcat_hbm.at[te_ref[i]], wbuf.at[s],
                              sems.at[s]).wait()

    @pl.when(i == 0)
    def _():
        start(te_ref[0], 0)

    @pl.when(jnp.logical_and(is_first, hn_ref[i] == 1))
    def _():
        @pl.when(slot == 0)
        def _():
            start(nxt_ref[i], 1)

        @pl.when(slot == 1)
        def _():
            start(nxt_ref[i], 0)

    @pl.when(is_first)
    def _():
        @pl.when(slot == 0)
        def _():
            wait(0)

        @pl.when(slot == 1)
        def _():
            wait(1)

    @pl.when(act)
    def _():
        x = xg_ref[...]
        acc = jnp.zeros((TILE, D), jnp.float32)
        for c in range(NCH):
            w1c = wbuf[slot, c * CH:(c + 1) * CH, :]
            w2c = wbuf[slot, DFF + c * CH:DFF + (c + 1) * CH, :]
            w3c = wbuf[slot, 2 * DFF + c * CH:2 * DFF + (c + 1) * CH, :]
            a = lax.dot_general(x, w1c, (((1,), (1,)), ((), ())),
                                preferred_element_type=jnp.float32)
            a = a + b1_ref[te_ref[i], :, c * CH:(c + 1) * CH]
            b = lax.dot_general(x, w2c, (((1,), (1,)), ((), ())),
                                preferred_element_type=jnp.float32)
            b = b + b2_ref[te_ref[i], :, c * CH:(c + 1) * CH]
            h = (a * jax.nn.sigmoid(a) * b).astype(jnp.bfloat16)
            acc = acc + lax.dot_general(h, w3c, (((1,), (0,)), ((), ())),
                                        preferred_element_type=jnp.float32)
        out_ref[...] = (acc + b3_ref[te_ref[i]]) * gw_ref[0]

    @pl.when(jnp.logical_not(act))
    def _():
        out_ref[...] = jnp.zeros_like(out_ref)


def _ffn(xg, Wcat, b1r, b2r, b3r, gw3, tile_e, tile_act,
         tile_ord, tile_nxt, tile_hn):
    grid_spec = pltpu.PrefetchScalarGridSpec(
        num_scalar_prefetch=5,
        grid=(NT,),
        in_specs=[
            pl.BlockSpec((TILE, D), lambda i, *_: (i, 0)),
            pl.BlockSpec(memory_space=pl.ANY),
            pl.BlockSpec((E, 1, DFF), lambda i, *_: (0, 0, 0)),
            pl.BlockSpec((E, 1, DFF), lambda i, *_: (0, 0, 0)),
            pl.BlockSpec((E, 1, D), lambda i, *_: (0, 0, 0)),
            pl.BlockSpec((1, TILE, 1), lambda i, *_: (i, 0, 0)),
        ],
        out_specs=pl.BlockSpec((TILE, D), lambda i, *_: (i, 0)),
        scratch_shapes=[
            pltpu.VMEM((2, 3 * DFF, D), jnp.bfloat16),
            pltpu.SemaphoreType.DMA((2,)),
        ],
    )
    return pl.pallas_call(
        _ffn_body,
        grid_spec=grid_spec,
        out_shape=jax.ShapeDtypeStruct((PAD_N, D), jnp.float32),
    )(tile_e, tile_act, tile_ord, tile_nxt, tile_hn,
      xg, Wcat, b1r, b2r, b3r, gw3)


_ADD_BLK = 512


def _add_body(a_ref, b_ref, o_ref):
    o_ref[...] = a_ref[...] + b_ref[...]


def _pair_add(zz):
    return pl.pallas_call(
        _add_body,
        grid=(T // _ADD_BLK,),
        in_specs=[
            pl.BlockSpec((_ADD_BLK, D), lambda i: (i, 0)),
            pl.BlockSpec((_ADD_BLK, D), lambda i: (T // _ADD_BLK + i, 0)),
        ],
        out_specs=pl.BlockSpec((_ADD_BLK, D), lambda i: (i, 0)),
        out_shape=jax.ShapeDtypeStruct((T, D), jnp.float32),
    )(zz, zz)


def kernel(x, W1, b1, W2, b2, W3, b3, Wg):
    x2d = x.reshape(T, D)
    topi, topw = _gate(x2d, Wg)
    (gidx, gw, p0, p1, tile_e, tile_act,
     tile_ord, tile_nxt, tile_hn) = _dispatch(topi, topw)
    xg = _sc_gather(x2d, gidx, PAD_N)
    Wcat = jnp.concatenate(
        [W1.astype(jnp.bfloat16), W2.astype(jnp.bfloat16),
         W3.transpose(0, 2, 1).astype(jnp.bfloat16)], axis=1)
    yw = _ffn(xg.astype(jnp.bfloat16), Wcat,
              b1.reshape(E, 1, DFF), b2.reshape(E, 1, DFF),
              b3.reshape(E, 1, D),
              gw.reshape(NT, TILE, 1), tile_e, tile_act,
              tile_ord, tile_nxt, tile_hn)
    zz = _sc_gather(yw, jnp.concatenate([p0, p1]), 2 * T)
    return _pair_add(zz).reshape(B, S, D)


# speedup vs baseline: 1.0003x; 1.0003x over previous
import functools

import jax
import jax.numpy as jnp
from jax import lax
from jax.experimental import pallas as pl
from jax.experimental.pallas import tpu as pltpu
from jax.experimental.pallas import tpu_sc as plsc

B, S, D, DFF, E, K = 2, 2048, 1024, 4096, 8, 2
T = B * S
A = T * K
TILE = 512
PAD_N = A + E * TILE
NT = PAD_N // TILE
CH = 512
NCH = DFF // CH

NC, NS = 2, 16
NW = NC * NS

GBLK = 1024


def _gate_body(x_ref, wg_ref, idx_ref, w_ref):
    logits = lax.dot_general(x_ref[...], wg_ref[...],
                             (((1,), (1,)), ((), ())),
                             preferred_element_type=jnp.float32)
    neg = jnp.full((GBLK, 1), -jnp.inf, jnp.float32)
    m1, i1 = neg, jnp.zeros((GBLK, 1), jnp.int32)
    for e in range(E):
        v = logits[:, e:e + 1]
        upd = v > m1
        m1 = jnp.where(upd, v, m1)
        i1 = jnp.where(upd, e, i1)
    m2, i2 = neg, jnp.zeros((GBLK, 1), jnp.int32)
    for e in range(E):
        v = logits[:, e:e + 1]
        upd = jnp.logical_and(v > m2, i1 != e)
        m2 = jnp.where(upd, v, m2)
        i2 = jnp.where(upd, e, i2)
    e2 = jnp.exp(m2 - m1)
    w1 = 1.0 / (1.0 + e2)
    idx_ref[...] = jnp.concatenate([i1, i2], axis=1)
    w_ref[...] = jnp.concatenate([w1, 1.0 - w1], axis=1)


def _gate(x2d, Wg):
    return pl.pallas_call(
        _gate_body,
        grid=(T // GBLK,),
        in_specs=[
            pl.BlockSpec((GBLK, D), lambda i: (i, 0)),
            pl.BlockSpec((E, D), lambda i: (0, 0)),
        ],
        out_specs=[
            pl.BlockSpec((GBLK, K), lambda i: (i, 0)),
            pl.BlockSpec((GBLK, K), lambda i: (i, 0)),
        ],
        out_shape=[
            jax.ShapeDtypeStruct((T, K), jnp.int32),
            jax.ShapeDtypeStruct((T, K), jnp.float32),
        ],
    )(x2d, Wg)


def _dispatch(topi, topw):
    ids = topi.reshape(-1)
    wts = topw.reshape(-1)
    onehot = (ids[:, None] == jnp.arange(E, dtype=jnp.int32)[None, :])
    csum = jnp.cumsum(onehot.astype(jnp.int32), axis=0)
    counts = csum[A - 1]
    pc = ((counts + TILE - 1) // TILE) * TILE
    pcc = jnp.cumsum(pc)
    poff = pcc - pc
    rank = jnp.take_along_axis(csum, ids[:, None], axis=1)[:, 0] - 1
    dest = jnp.take(poff, ids) + rank
    ar = jnp.arange(A, dtype=jnp.int32)
    gidx = (jnp.arange(PAD_N, dtype=jnp.int32) % T).at[dest].set(ar // K)
    gw = jnp.zeros((PAD_N,), jnp.float32).at[dest].set(wts)
    p0, p1 = dest[0::2], dest[1::2]
    tile_start = jnp.arange(NT, dtype=jnp.int32) * TILE
    tile_e = jnp.minimum(
        jnp.searchsorted(pcc, tile_start, side="right").astype(jnp.int32),
        E - 1)
    tile_act = (tile_start < pcc[E - 1]).astype(jnp.int32)
    chg = (tile_e[1:] != tile_e[:-1]).astype(jnp.int32)
    tile_ord = jnp.cumsum(jnp.concatenate([jnp.zeros((1,), jnp.int32), chg]))
    nxt_pos = jnp.searchsorted(tile_e, tile_e, side="right").astype(jnp.int32)
    tile_hn = (nxt_pos < NT).astype(jnp.int32)
    tile_nxt = jnp.take(tile_e, jnp.minimum(nxt_pos, NT - 1))
    return (gidx, gw, p0, p1, tile_e, tile_act, tile_ord, tile_nxt, tile_hn)


_G_CHUNK = 32


def _gather_body(n_rows, tab_hbm, idx_hbm, out_hbm,
                 i0, i1, r0, r1, sem):
    per_w = n_rows // NW
    nch = per_w // _G_CHUNK
    wid = lax.axis_index("s") * NC + lax.axis_index("c")
    base = wid * per_w
    bufs = ((i0, r0), (i1, r1))

    pltpu.sync_copy(idx_hbm.at[pl.ds(base, _G_CHUNK)], i0)
    cp = pltpu.async_copy(tab_hbm.at[i0], r0, sem)
    for c in range(nch):
        _, rb = bufs[c % 2]
        if c + 1 < nch:
            inx, rnx = bufs[(c + 1) % 2]
            pltpu.sync_copy(
                idx_hbm.at[pl.ds(base + (c + 1) * _G_CHUNK, _G_CHUNK)], inx)
            cpn = pltpu.async_copy(tab_hbm.at[inx], rnx, sem)
        cp.wait()
        pltpu.sync_copy(rb, out_hbm.at[pl.ds(base + c * _G_CHUNK, _G_CHUNK)])
        if c + 1 < nch:
            cp = cpn


@functools.cache
def _sc_gather_kernel(n_rows):
    return pl.kernel(
        functools.partial(_gather_body, n_rows),
        name=f"sc_row_gather_{n_rows}",
        out_type=jax.ShapeDtypeStruct((n_rows, D), jnp.float32),
        mesh=plsc.VectorSubcoreMesh(core_axis_name="c", subcore_axis_name="s",
                                    num_cores=NC, num_subcores=NS),
        scratch_types=[
            pltpu.VMEM((_G_CHUNK,), jnp.int32),
            pltpu.VMEM((_G_CHUNK,), jnp.int32),
            pltpu.VMEM((_G_CHUNK, D), jnp.float32),
            pltpu.VMEM((_G_CHUNK, D), jnp.float32),
            pltpu.SemaphoreType.DMA,
        ],
    )


def _sc_gather(tab, idx, n_rows):
    return _sc_gather_kernel(n_rows)(tab, idx)


def _ffn_body(te_ref, ta_ref, ord_ref, nxt_ref, hn_ref,
              xg_ref, wcat_hbm, b1_ref, b2_ref, b3_ref,
              gw_ref, out_ref, wbuf, sems):
    i = pl.program_id(0)
    act = ta_ref[i] == 1
    slot = ord_ref[i] % 2
    prev_ord = ord_ref[jnp.maximum(i - 1, 0)]
    is_first = jnp.logical_or(i == 0, ord_ref[i] != prev_ord)

    NSPLIT = 12
    SL = (3 * DFF) // NSPLIT

    def start(e_idx, s):
        for k in range(NSPLIT):
            pltpu.make_async_copy(
                wcat_hbm.at[e_idx, pl.ds(k * SL, SL)],
                wbuf.at[s, pl.ds(k * SL, SL)],
                sems.at[s]).start()

    def wait(s):
        for k in range(NSPLIT):
            pltpu.make_async_copy(
                wcat_hbm.at[te_ref[i], pl.ds(k * SL, SL)],
                wbuf.at[s, pl.ds(k * SL, SL)],
                sems.at[s]).wait()

    @pl.when(i == 0)
    def _():
        start(te_ref[0], 0)

    @pl.when(jnp.logical_and(is_first, hn_ref[i] == 1))
    def _():
        @pl.when(slot == 0)
        def _():
            start(nxt_ref[i], 1)

        @pl.when(slot == 1)
        def _():
            start(nxt_ref[i], 0)

    @pl.when(is_first)
    def _():
        @pl.when(slot == 0)
        def _():
            wait(0)

        @pl.when(slot == 1)
        def _():
            wait(1)

    @pl.when(act)
    def _():
        x = xg_ref[...]
        acc = jnp.zeros((TILE, D), jnp.float32)
        for c in range(NCH):
            w1c = wbuf[slot, c * CH:(c + 1) * CH, :]
            w2c = wbuf[slot, DFF + c * CH:DFF + (c + 1) * CH, :]
            w3c = wbuf[slot, 2 * DFF + c * CH:2 * DFF + (c + 1) * CH, :]
            a = lax.dot_general(x, w1c, (((1,), (1,)), ((), ())),
                                preferred_element_type=jnp.float32)
            a = a + b1_ref[te_ref[i], :, c * CH:(c + 1) * CH]
            b = lax.dot_general(x, w2c, (((1,), (1,)), ((), ())),
                                preferred_element_type=jnp.float32)
            b = b + b2_ref[te_ref[i], :, c * CH:(c + 1) * CH]
            h = (a * jax.nn.sigmoid(a) * b).astype(jnp.bfloat16)
            acc = acc + lax.dot_general(h, w3c, (((1,), (0,)), ((), ())),
                                        preferred_element_type=jnp.float32)
        out_ref[...] = (acc + b3_ref[te_ref[i]]) * gw_ref[0]

    @pl.when(jnp.logical_not(act))
    def _():
        out_ref[...] = jnp.zeros_like(out_ref)


def _ffn(xg, Wcat, b1r, b2r, b3r, gw3, tile_e, tile_act,
         tile_ord, tile_nxt, tile_hn):
    grid_spec = pltpu.PrefetchScalarGridSpec(
        num_scalar_prefetch=5,
        grid=(NT,),
        in_specs=[
            pl.BlockSpec((TILE, D), lambda i, *_: (i, 0)),
            pl.BlockSpec(memory_space=pl.ANY),
            pl.BlockSpec((E, 1, DFF), lambda i, *_: (0, 0, 0)),
            pl.BlockSpec((E, 1, DFF), lambda i, *_: (0, 0, 0)),
            pl.BlockSpec((E, 1, D), lambda i, *_: (0, 0, 0)),
            pl.BlockSpec((1, TILE, 1), lambda i, *_: (i, 0, 0)),
        ],
        out_specs=pl.BlockSpec((TILE, D), lambda i, *_: (i, 0)),
        scratch_shapes=[
            pltpu.VMEM((2, 3 * DFF, D), jnp.bfloat16),
            pltpu.SemaphoreType.DMA((2,)),
        ],
    )
    return pl.pallas_call(
        _ffn_body,
        grid_spec=grid_spec,
        out_shape=jax.ShapeDtypeStruct((PAD_N, D), jnp.float32),
    )(tile_e, tile_act, tile_ord, tile_nxt, tile_hn,
      xg, Wcat, b1r, b2r, b3r, gw3)


_ADD_BLK = 512


def _add_body(a_ref, b_ref, o_ref):
    o_ref[...] = a_ref[...] + b_ref[...]


def _pair_add(zz):
    return pl.pallas_call(
        _add_body,
        grid=(T // _ADD_BLK,),
        in_specs=[
            pl.BlockSpec((_ADD_BLK, D), lambda i: (i, 0)),
            pl.BlockSpec((_ADD_BLK, D), lambda i: (T // _ADD_BLK + i, 0)),
        ],
        out_specs=pl.BlockSpec((_ADD_BLK, D), lambda i: (i, 0)),
        out_shape=jax.ShapeDtypeStruct((T, D), jnp.float32),
    )(zz, zz)


def kernel(x, W1, b1, W2, b2, W3, b3, Wg):
    x2d = x.reshape(T, D)
    topi, topw = _gate(x2d, Wg)
    (gidx, gw, p0, p1, tile_e, tile_act,
     tile_ord, tile_nxt, tile_hn) = _dispatch(topi, topw)
    xg = _sc_gather(x2d, gidx, PAD_N)
    Wcat = jnp.concatenate(
        [W1.astype(jnp.bfloat16), W2.astype(jnp.bfloat16),
         W3.transpose(0, 2, 1).astype(jnp.bfloat16)], axis=1)
    yw = _ffn(xg.astype(jnp.bfloat16), Wcat,
              b1.reshape(E, 1, DFF), b2.reshape(E, 1, DFF),
              b3.reshape(E, 1, D),
              gw.reshape(NT, TILE, 1), tile_e, tile_act,
              tile_ord, tile_nxt, tile_hn)
    zz = _sc_gather(yw, jnp.concatenate([p0, p1]), 2 * T)
    return _pair_add(zz).reshape(B, S, D)


# speedup vs baseline: 1.1992x; 1.1988x over previous
import functools

import jax
import jax.numpy as jnp
from jax import lax
from jax.experimental import pallas as pl
from jax.experimental.pallas import tpu as pltpu
from jax.experimental.pallas import tpu_sc as plsc

B, S, D, DFF, E, K = 2, 2048, 1024, 4096, 8, 2
T = B * S
A = T * K
TILE = 1024
PAD_N = A + E * TILE
NT = PAD_N // TILE
DBLK = 1024
NDB = DFF // DBLK

NC, NS = 2, 16
NW = NC * NS

GBLK = 1024


def _gate_body(x_ref, wg_ref, idx_ref, w_ref):
    logits = lax.dot_general(x_ref[...], wg_ref[...],
                             (((1,), (1,)), ((), ())),
                             preferred_element_type=jnp.float32)
    neg = jnp.full((GBLK, 1), -jnp.inf, jnp.float32)
    m1, i1 = neg, jnp.zeros((GBLK, 1), jnp.int32)
    for e in range(E):
        v = logits[:, e:e + 1]
        upd = v > m1
        m1 = jnp.where(upd, v, m1)
        i1 = jnp.where(upd, e, i1)
    m2, i2 = neg, jnp.zeros((GBLK, 1), jnp.int32)
    for e in range(E):
        v = logits[:, e:e + 1]
        upd = jnp.logical_and(v > m2, i1 != e)
        m2 = jnp.where(upd, v, m2)
        i2 = jnp.where(upd, e, i2)
    e2 = jnp.exp(m2 - m1)
    w1 = 1.0 / (1.0 + e2)
    idx_ref[...] = jnp.concatenate([i1, i2], axis=1)
    w_ref[...] = jnp.concatenate([w1, 1.0 - w1], axis=1)


def _gate(x2d, Wg):
    return pl.pallas_call(
        _gate_body,
        grid=(T // GBLK,),
        in_specs=[
            pl.BlockSpec((GBLK, D), lambda i: (i, 0)),
            pl.BlockSpec((E, D), lambda i: (0, 0)),
        ],
        out_specs=[
            pl.BlockSpec((GBLK, K), lambda i: (i, 0)),
            pl.BlockSpec((GBLK, K), lambda i: (i, 0)),
        ],
        out_shape=[
            jax.ShapeDtypeStruct((T, K), jnp.int32),
            jax.ShapeDtypeStruct((T, K), jnp.float32),
        ],
    )(x2d, Wg)


def _dispatch(topi, topw):
    ids = topi.reshape(-1)
    wts = topw.reshape(-1)
    onehot = (ids[:, None] == jnp.arange(E, dtype=jnp.int32)[None, :])
    csum = jnp.cumsum(onehot.astype(jnp.int32), axis=0)
    counts = csum[A - 1]
    pc = ((counts + TILE - 1) // TILE) * TILE
    pcc = jnp.cumsum(pc)
    poff = pcc - pc
    rank = jnp.take_along_axis(csum, ids[:, None], axis=1)[:, 0] - 1
    dest = jnp.take(poff, ids) + rank
    ar = jnp.arange(A, dtype=jnp.int32)
    gidx = (jnp.arange(PAD_N, dtype=jnp.int32) % T).at[dest].set(ar // K)
    gw = jnp.zeros((PAD_N,), jnp.float32).at[dest].set(wts)
    p0, p1 = dest[0::2], dest[1::2]
    tile_start = jnp.arange(NT, dtype=jnp.int32) * TILE
    tile_e = jnp.minimum(
        jnp.searchsorted(pcc, tile_start, side="right").astype(jnp.int32),
        E - 1)
    tile_act = (tile_start < pcc[E - 1]).astype(jnp.int32)
    return gidx, gw, p0, p1, tile_e, tile_act


_G_CHUNK = 32


def _gather_body(n_rows, tab_hbm, idx_hbm, out_hbm,
                 i0, i1, r0, r1, sem):
    per_w = n_rows // NW
    nch = per_w // _G_CHUNK
    wid = lax.axis_index("s") * NC + lax.axis_index("c")
    base = wid * per_w
    bufs = ((i0, r0), (i1, r1))

    pltpu.sync_copy(idx_hbm.at[pl.ds(base, _G_CHUNK)], i0)
    cp = pltpu.async_copy(tab_hbm.at[i0], r0, sem)
    for c in range(nch):
        _, rb = bufs[c % 2]
        if c + 1 < nch:
            inx, rnx = bufs[(c + 1) % 2]
            pltpu.sync_copy(
                idx_hbm.at[pl.ds(base + (c + 1) * _G_CHUNK, _G_CHUNK)], inx)
            cpn = pltpu.async_copy(tab_hbm.at[inx], rnx, sem)
        cp.wait()
        pltpu.sync_copy(rb, out_hbm.at[pl.ds(base + c * _G_CHUNK, _G_CHUNK)])
        if c + 1 < nch:
            cp = cpn


@functools.cache
def _sc_gather_kernel(n_rows):
    return pl.kernel(
        functools.partial(_gather_body, n_rows),
        name=f"sc_row_gather_{n_rows}",
        out_type=jax.ShapeDtypeStruct((n_rows, D), jnp.float32),
        mesh=plsc.VectorSubcoreMesh(core_axis_name="c", subcore_axis_name="s",
                                    num_cores=NC, num_subcores=NS),
        scratch_types=[
            pltpu.VMEM((_G_CHUNK,), jnp.int32),
            pltpu.VMEM((_G_CHUNK,), jnp.int32),
            pltpu.VMEM((_G_CHUNK, D), jnp.float32),
            pltpu.VMEM((_G_CHUNK, D), jnp.float32),
            pltpu.SemaphoreType.DMA,
        ],
    )


def _sc_gather(tab, idx, n_rows):
    return _sc_gather_kernel(n_rows)(tab, idx)


def _ffn_body(te_ref, ta_ref, xg_ref, w1_ref, w2_ref, w3_ref,
              b1_ref, b2_ref, b3_ref, gw_ref, out_ref, acc_ref):
    i = pl.program_id(0)
    j = pl.program_id(1)
    act = ta_ref[i] == 1

    @pl.when(act)
    def _():
        x = xg_ref[...].astype(jnp.bfloat16)
        a = lax.dot_general(x, w1_ref[0], (((1,), (1,)), ((), ())),
                            preferred_element_type=jnp.float32) + b1_ref[0]
        b = lax.dot_general(x, w2_ref[0], (((1,), (1,)), ((), ())),
                            preferred_element_type=jnp.float32) + b2_ref[0]
        h = (a * jax.nn.sigmoid(a) * b).astype(jnp.bfloat16)
        y = lax.dot_general(h, w3_ref[0], (((1,), (1,)), ((), ())),
                            preferred_element_type=jnp.float32)

        @pl.when(j == 0)
        def _():
            acc_ref[...] = y

        @pl.when(j > 0)
        def _():
            acc_ref[...] += y

    @pl.when(j == NDB - 1)
    def _():
        @pl.when(act)
        def _():
            out_ref[...] = (acc_ref[...] + b3_ref[0]) * gw_ref[0]

        @pl.when(jnp.logical_not(act))
        def _():
            out_ref[...] = jnp.zeros_like(out_ref)


def _ffn(xg, W1b, W2b, W3b, b1r, b2r, b3r, gw3, tile_e, tile_act):
    grid_spec = pltpu.PrefetchScalarGridSpec(
        num_scalar_prefetch=2,
        grid=(NT, NDB),
        in_specs=[
            pl.BlockSpec((TILE, D), lambda i, j, te, ta: (i, 0)),
            pl.BlockSpec((1, DBLK, D), lambda i, j, te, ta: (te[i], j, 0)),
            pl.BlockSpec((1, DBLK, D), lambda i, j, te, ta: (te[i], j, 0)),
            pl.BlockSpec((1, D, DBLK), lambda i, j, te, ta: (te[i], 0, j)),
            pl.BlockSpec((1, 1, DBLK), lambda i, j, te, ta: (te[i], 0, j)),
            pl.BlockSpec((1, 1, DBLK), lambda i, j, te, ta: (te[i], 0, j)),
            pl.BlockSpec((1, 1, D), lambda i, j, te, ta: (te[i], 0, 0)),
            pl.BlockSpec((1, TILE, 1), lambda i, j, te, ta: (i, 0, 0)),
        ],
        out_specs=pl.BlockSpec((TILE, D), lambda i, j, te, ta: (i, 0)),
        scratch_shapes=[pltpu.VMEM((TILE, D), jnp.float32)],
    )
    return pl.pallas_call(
        _ffn_body,
        grid_spec=grid_spec,
        out_shape=jax.ShapeDtypeStruct((PAD_N, D), jnp.float32),
    )(tile_e, tile_act, xg, W1b, W2b, W3b, b1r, b2r, b3r, gw3)


_ADD_BLK = 512


def _add_body(a_ref, b_ref, o_ref):
    o_ref[...] = a_ref[...] + b_ref[...]


def _pair_add(zz):
    return pl.pallas_call(
        _add_body,
        grid=(T // _ADD_BLK,),
        in_specs=[
            pl.BlockSpec((_ADD_BLK, D), lambda i: (i, 0)),
            pl.BlockSpec((_ADD_BLK, D), lambda i: (T // _ADD_BLK + i, 0)),
        ],
        out_specs=pl.BlockSpec((_ADD_BLK, D), lambda i: (i, 0)),
        out_shape=jax.ShapeDtypeStruct((T, D), jnp.float32),
    )(zz, zz)


def kernel(x, W1, b1, W2, b2, W3, b3, Wg):
    x2d = x.reshape(T, D)
    topi, topw = _gate(x2d, Wg)
    gidx, gw, p0, p1, tile_e, tile_act = _dispatch(topi, topw)
    xg = _sc_gather(x2d, gidx, PAD_N)
    yw = _ffn(xg,
              W1.astype(jnp.bfloat16),
              W2.astype(jnp.bfloat16),
              W3.astype(jnp.bfloat16),
              b1.reshape(E, 1, DFF), b2.reshape(E, 1, DFF),
              b3.reshape(E, 1, D),
              gw.reshape(NT, TILE, 1), tile_e, tile_act)
    zz = _sc_gather(yw, jnp.concatenate([p0, p1]), 2 * T)
    return _pair_add(zz).reshape(B, S, D)


# speedup vs baseline: 1.3384x; 1.1161x over previous
import functools

import jax
import jax.numpy as jnp
from jax import lax
from jax.experimental import pallas as pl
from jax.experimental.pallas import tpu as pltpu
from jax.experimental.pallas import tpu_sc as plsc

B, S, D, DFF, E, K = 2, 2048, 1024, 4096, 8, 2
T = B * S
A = T * K
TILE = 512
PAD_N = A + E * TILE
NT = PAD_N // TILE
DBLK = 2048
NDB = DFF // DBLK

NC, NS = 2, 16
NW = NC * NS

GBLK = 1024


def _gate_body(x_ref, wg_ref, idx_ref, w_ref):
    logits = lax.dot_general(x_ref[...], wg_ref[...],
                             (((1,), (1,)), ((), ())),
                             preferred_element_type=jnp.float32)
    neg = jnp.full((GBLK, 1), -jnp.inf, jnp.float32)
    m1, i1 = neg, jnp.zeros((GBLK, 1), jnp.int32)
    for e in range(E):
        v = logits[:, e:e + 1]
        upd = v > m1
        m1 = jnp.where(upd, v, m1)
        i1 = jnp.where(upd, e, i1)
    m2, i2 = neg, jnp.zeros((GBLK, 1), jnp.int32)
    for e in range(E):
        v = logits[:, e:e + 1]
        upd = jnp.logical_and(v > m2, i1 != e)
        m2 = jnp.where(upd, v, m2)
        i2 = jnp.where(upd, e, i2)
    e2 = jnp.exp(m2 - m1)
    w1 = 1.0 / (1.0 + e2)
    idx_ref[...] = jnp.concatenate([i1, i2], axis=1)
    w_ref[...] = jnp.concatenate([w1, 1.0 - w1], axis=1)


def _gate(x2d, Wg):
    return pl.pallas_call(
        _gate_body,
        grid=(T // GBLK,),
        in_specs=[
            pl.BlockSpec((GBLK, D), lambda i: (i, 0)),
            pl.BlockSpec((E, D), lambda i: (0, 0)),
        ],
        out_specs=[
            pl.BlockSpec((GBLK, K), lambda i: (i, 0)),
            pl.BlockSpec((GBLK, K), lambda i: (i, 0)),
        ],
        out_shape=[
            jax.ShapeDtypeStruct((T, K), jnp.int32),
            jax.ShapeDtypeStruct((T, K), jnp.float32),
        ],
    )(x2d, Wg)


def _dispatch(topi, topw):
    ids = topi.reshape(-1)
    wts = topw.reshape(-1)
    onehot = (ids[:, None] == jnp.arange(E, dtype=jnp.int32)[None, :])
    csum = jnp.cumsum(onehot.astype(jnp.int32), axis=0)
    counts = csum[A - 1]
    pc = ((counts + TILE - 1) // TILE) * TILE
    pcc = jnp.cumsum(pc)
    poff = pcc - pc
    rank = jnp.take_along_axis(csum, ids[:, None], axis=1)[:, 0] - 1
    dest = jnp.take(poff, ids) + rank
    ar = jnp.arange(A, dtype=jnp.int32)
    gidx = (jnp.arange(PAD_N, dtype=jnp.int32) % T).at[dest].set(ar // K)
    gw = jnp.zeros((PAD_N,), jnp.float32).at[dest].set(wts)
    p0, p1 = dest[0::2], dest[1::2]
    tile_start = jnp.arange(NT, dtype=jnp.int32) * TILE
    tile_e = jnp.minimum(
        jnp.searchsorted(pcc, tile_start, side="right").astype(jnp.int32),
        E - 1)
    tile_act = (tile_start < pcc[E - 1]).astype(jnp.int32)
    return gidx, gw, p0, p1, tile_e, tile_act


_G_CHUNK = 32


def _gather_body(n_rows, tab_hbm, idx_hbm, out_hbm,
                 i0, i1, r0, r1, sem):
    per_w = n_rows // NW
    nch = per_w // _G_CHUNK
    wid = lax.axis_index("s") * NC + lax.axis_index("c")
    base = wid * per_w
    bufs = ((i0, r0), (i1, r1))

    pltpu.sync_copy(idx_hbm.at[pl.ds(base, _G_CHUNK)], i0)
    cp = pltpu.async_copy(tab_hbm.at[i0], r0, sem)
    for c in range(nch):
        _, rb = bufs[c % 2]
        if c + 1 < nch:
            inx, rnx = bufs[(c + 1) % 2]
            pltpu.sync_copy(
                idx_hbm.at[pl.ds(base + (c + 1) * _G_CHUNK, _G_CHUNK)], inx)
            cpn = pltpu.async_copy(tab_hbm.at[inx], rnx, sem)
        cp.wait()
        pltpu.sync_copy(rb, out_hbm.at[pl.ds(base + c * _G_CHUNK, _G_CHUNK)])
        if c + 1 < nch:
            cp = cpn


@functools.cache
def _sc_gather_kernel(n_rows):
    return pl.kernel(
        functools.partial(_gather_body, n_rows),
        name=f"sc_row_gather_{n_rows}",
        out_type=jax.ShapeDtypeStruct((n_rows, D), jnp.float32),
        mesh=plsc.VectorSubcoreMesh(core_axis_name="c", subcore_axis_name="s",
                                    num_cores=NC, num_subcores=NS),
        scratch_types=[
            pltpu.VMEM((_G_CHUNK,), jnp.int32),
            pltpu.VMEM((_G_CHUNK,), jnp.int32),
            pltpu.VMEM((_G_CHUNK, D), jnp.float32),
            pltpu.VMEM((_G_CHUNK, D), jnp.float32),
            pltpu.SemaphoreType.DMA,
        ],
    )


def _sc_gather(tab, idx, n_rows):
    return _sc_gather_kernel(n_rows)(tab, idx)


def _ffn_body(te_ref, ta_ref, xg_ref, w1_ref, w2_ref, w3_ref,
              b1_ref, b2_ref, b3_ref, gw_ref, out_ref, acc_ref):
    i = pl.program_id(0)
    j = pl.program_id(1)
    act = ta_ref[i] == 1

    @pl.when(act)
    def _():
        x = xg_ref[...].astype(jnp.bfloat16)
        a = lax.dot_general(x, w1_ref[0], (((1,), (1,)), ((), ())),
                            preferred_element_type=jnp.float32) + b1_ref[0]
        b = lax.dot_general(x, w2_ref[0], (((1,), (1,)), ((), ())),
                            preferred_element_type=jnp.float32) + b2_ref[0]
        h = (a * jax.nn.sigmoid(a) * b).astype(jnp.bfloat16)
        y = lax.dot_general(h, w3_ref[0], (((1,), (1,)), ((), ())),
                            preferred_element_type=jnp.float32)

        @pl.when(j == 0)
        def _():
            acc_ref[...] = y

        @pl.when(j > 0)
        def _():
            acc_ref[...] += y

    @pl.when(j == NDB - 1)
    def _():
        @pl.when(act)
        def _():
            out_ref[...] = (acc_ref[...] + b3_ref[0]) * gw_ref[0]

        @pl.when(jnp.logical_not(act))
        def _():
            out_ref[...] = jnp.zeros_like(out_ref)


def _ffn(xg, W1b, W2b, W3b, b1r, b2r, b3r, gw3, tile_e, tile_act):
    grid_spec = pltpu.PrefetchScalarGridSpec(
        num_scalar_prefetch=2,
        grid=(NT, NDB),
        in_specs=[
            pl.BlockSpec((TILE, D), lambda i, j, te, ta: (i, 0)),
            pl.BlockSpec((1, DBLK, D), lambda i, j, te, ta: (te[i], j, 0)),
            pl.BlockSpec((1, DBLK, D), lambda i, j, te, ta: (te[i], j, 0)),
            pl.BlockSpec((1, D, DBLK), lambda i, j, te, ta: (te[i], 0, j)),
            pl.BlockSpec((1, 1, DBLK), lambda i, j, te, ta: (te[i], 0, j)),
            pl.BlockSpec((1, 1, DBLK), lambda i, j, te, ta: (te[i], 0, j)),
            pl.BlockSpec((1, 1, D), lambda i, j, te, ta: (te[i], 0, 0)),
            pl.BlockSpec((1, TILE, 1), lambda i, j, te, ta: (i, 0, 0)),
        ],
        out_specs=pl.BlockSpec((TILE, D), lambda i, j, te, ta: (i, 0)),
        scratch_shapes=[pltpu.VMEM((TILE, D), jnp.float32)],
    )
    return pl.pallas_call(
        _ffn_body,
        grid_spec=grid_spec,
        out_shape=jax.ShapeDtypeStruct((PAD_N, D), jnp.float32),
    )(tile_e, tile_act, xg, W1b, W2b, W3b, b1r, b2r, b3r, gw3)


_ADD_BLK = 512


def _add_body(a_ref, b_ref, o_ref):
    o_ref[...] = a_ref[...] + b_ref[...]


def _pair_add(zz):
    return pl.pallas_call(
        _add_body,
        grid=(T // _ADD_BLK,),
        in_specs=[
            pl.BlockSpec((_ADD_BLK, D), lambda i: (i, 0)),
            pl.BlockSpec((_ADD_BLK, D), lambda i: (T // _ADD_BLK + i, 0)),
        ],
        out_specs=pl.BlockSpec((_ADD_BLK, D), lambda i: (i, 0)),
        out_shape=jax.ShapeDtypeStruct((T, D), jnp.float32),
    )(zz, zz)


def kernel(x, W1, b1, W2, b2, W3, b3, Wg):
    x2d = x.reshape(T, D)
    topi, topw = _gate(x2d, Wg)
    gidx, gw, p0, p1, tile_e, tile_act = _dispatch(topi, topw)
    xg = _sc_gather(x2d, gidx, PAD_N)
    yw = _ffn(xg,
              W1.astype(jnp.bfloat16),
              W2.astype(jnp.bfloat16),
              W3.astype(jnp.bfloat16),
              b1.reshape(E, 1, DFF), b2.reshape(E, 1, DFF),
              b3.reshape(E, 1, D),
              gw.reshape(NT, TILE, 1), tile_e, tile_act)
    zz = _sc_gather(yw, jnp.concatenate([p0, p1]), 2 * T)
    return _pair_add(zz).reshape(B, S, D)
